# baseline (speedup 1.0000x reference)
import functools

import jax
import jax.numpy as jnp
from jax.experimental import pallas as pl

_ALPHA = 0.007
_BETA = 0.016
_OMEGA_O = 0.57
_OMEGA_T = 0.41
_NO_VALUE = -999.0
_EDGE_W = (1.0, 0.98, 0.97, 0.96, 0.95)

_B, _T, _C, _H, _W = 4, 12, 1, 512, 512
_NUMEL = _B * _C * _H * _W
_K90_LO = 943717
_K13 = 349525

_IMIN = -2147483648


def _f32_key(x):
    u = jax.lax.bitcast_convert_type(x, jnp.int32)
    return jnp.where(u >= 0, u, _IMIN - u)


def _key_to_f32(k):
    u = jnp.where(k >= 0, k, _IMIN - k)
    return jax.lax.bitcast_convert_type(u.astype(jnp.int32), jnp.float32)


def _select_rank(key, k90, k13):
    imin = jnp.int32(_IMIN)
    kmin = jnp.min(key)
    kmax = jnp.max(key)
    n = jnp.int32(_NUMEL)

    def init(k):
        return (kmin - 1, kmax, jnp.int32(0), n,
                _key_to_f32(kmin), _key_to_f32(kmax))

    def propose(st, k, use_mid):
        lo, hi, clo, chi, vlo, vhi = st
        gap = hi - lo
        half = jax.lax.shift_right_logical(gap, 1)
        mid = jax.lax.bitwise_xor(
            jax.lax.bitwise_xor(lo, imin) + half, imin)
        frac = (jnp.float32(k + 1) - clo.astype(jnp.float32)) / (
            chi.astype(jnp.float32) - clo.astype(jnp.float32))
        tv = vlo + (vhi - vlo) * frac
        tu = jax.lax.bitcast_convert_type(tv, jnp.int32)
        tk = jnp.where(tu >= 0, tu, imin - tu)
        tk = jnp.minimum(jnp.maximum(tk, lo + 1), hi - 1)
        return jnp.where(use_mid, mid, tk)

    def update(st, k, t, cnt):
        lo, hi, clo, chi, vlo, vhi = st
        active = (hi - lo) != 1
        up = active & (cnt >= k + 1)
        dn = active & (cnt < k + 1)
        tval = _key_to_f32(t)
        return (jnp.where(dn, t, lo), jnp.where(up, t, hi),
                jnp.where(dn, cnt, clo), jnp.where(up, cnt, chi),
                jnp.where(dn, tval, vlo), jnp.where(up, tval, vhi))

    def cond(carry):
        r, st90, st13 = carry
        return ((st90[1] - st90[0]) != 1) | ((st13[1] - st13[0]) != 1)

    def body(carry):
        r, st90, st13 = carry
        use_mid = jax.lax.bitwise_and(r, jnp.int32(1)) == 1
        t90 = propose(st90, k90, use_mid)
        t13 = propose(st13, k13, use_mid)
        c90 = jnp.sum((key <= t90).astype(jnp.int32))
        c13 = jnp.sum((key <= t13).astype(jnp.int32))
        return (r + 1, update(st90, k90, t90, c90),
                update(st13, k13, t13, c13))

    carry = (jnp.int32(0), init(k90), init(k13))
    _, st90, st13 = jax.lax.while_loop(cond, body, carry)
    return st90[1], st13[1]


def _edge_weight(idx):
    e = jnp.zeros_like(idx, dtype=jnp.float32)
    for i, w in enumerate(_EDGE_W):
        e = e + jnp.where(idx == i, w, 0.0) + jnp.where(idx == (_H - 1 - i), w, 0.0)
    return e


def _min_weight(m):
    e = jnp.zeros_like(m, dtype=jnp.float32)
    for i, w in enumerate(_EDGE_W):
        e = e + jnp.where(m == i, w, 0.0)
    return e


def _loss_kernel(yp_ref, yt_ref, out_ref):
    yt = yt_ref[...]
    yp = yp_ref[...]

    key = _f32_key(yt)
    key90a, key13 = _select_rank(key, _K90_LO, _K13)
    c_a = jnp.sum((key <= key90a).astype(jnp.int32))
    nxt = jnp.min(jnp.where(key > key90a, key, jnp.int32(2147483647)))
    key90b = jnp.where(c_a >= _K90_LO + 2, key90a, nxt)
    va = _key_to_f32(key90a)
    vb = _key_to_f32(key90b)
    q90 = va + (vb - va) * jnp.float32(0.5)
    q13 = _key_to_f32(key13)

    h = jax.lax.broadcasted_iota(jnp.int32, yt.shape, 3)
    w = jax.lax.broadcasted_iota(jnp.int32, yt.shape, 4)
    diag = (h == w) | (h + w == _H - 1)
    bmask = _edge_weight(h) + _edge_weight(w) + jnp.where(
        diag, _min_weight(jnp.minimum(h, _H - 1 - h)), 0.0)

    diff = jnp.abs(yt - yp)
    no_value = yt == _NO_VALUE
    outlier = yt > q90
    normal = jnp.logical_not(no_value | outlier)
    over = (yp >= yt) & normal
    under = (yp < yt) & normal
    torr = (yt >= q13) & normal
    wts = _ALPHA * jnp.exp(_BETA * yt)
    sq = (yt - yp) * (yt - yp)
    wsq = wts * sq

    zero = jnp.float32(0.0)
    s_low = jnp.sum(jnp.where(no_value, diff, zero))
    s_out = jnp.sum(jnp.where(outlier, diff, zero))
    s_bnd = jnp.sum(bmask * diff)
    s_over = jnp.sum(jnp.where(over, diff, zero))
    s_under = jnp.sum(jnp.where(under, diff, zero))
    s_tover = jnp.sum(jnp.where(torr & over, wsq, zero))
    s_tunder = jnp.sum(jnp.where(torr & under, wsq, zero))
    s_abs = jnp.sum(jnp.where(no_value, zero, diff))
    s_cnt = jnp.sum(jnp.where(no_value, zero, jnp.float32(1.0)))

    inv_n = jnp.float32(1.0 / _NUMEL)
    loss_t = (
        _OMEGA_O * s_low
        + (1.0 - _OMEGA_O) * s_out
        + (1.0 - _OMEGA_O) * s_bnd
        + (1.0 - _OMEGA_O) * s_over + _OMEGA_O * s_under
        + (1.0 - _OMEGA_T) * s_tover + _OMEGA_T * s_tunder
    ) * inv_n

    r = jax.lax.broadcasted_iota(jnp.int32, (1, 8, 128), 1)
    c = jax.lax.broadcasted_iota(jnp.int32, (1, 8, 128), 2)
    first = r == 0
    tile = (jnp.where(first & (c == 0), loss_t, zero)
            + jnp.where(first & (c == 1), s_abs, zero)
            + jnp.where(first & (c == 2), s_cnt, zero))
    out_ref[...] = tile


@jax.jit
def kernel(y_pred, y_true):
    block = (_B, 1, _C, _H, _W)
    partials = pl.pallas_call(
        _loss_kernel,
        grid=(_T,),
        in_specs=[
            pl.BlockSpec(block, lambda t: (0, t, 0, 0, 0)),
            pl.BlockSpec(block, lambda t: (0, t, 0, 0, 0)),
        ],
        out_specs=pl.BlockSpec((1, 8, 128), lambda t: (t, 0, 0)),
        out_shape=jax.ShapeDtypeStruct((_T, 8, 128), jnp.float32),
    )(y_pred, y_true)

    losses = partials[:, 0, 0]
    s_abs = partials[:, 0, 1]
    s_cnt = partials[:, 0, 2]
    seasons = ((0, 1, 11), (2, 3, 4), (5, 6, 7), (8, 9, 10))
    seasonal = jnp.float32(0.0)
    for idx in seasons:
        ii = jnp.asarray(idx)
        seasonal = seasonal + jnp.sum(s_abs[ii]) / jnp.sum(s_cnt[ii])
    return jnp.mean(losses) + seasonal


# speedup vs baseline: 2.5408x; 2.5408x over previous
import functools

import jax
import jax.numpy as jnp
from jax.experimental import pallas as pl

_ALPHA = 0.007
_BETA = 0.016
_OMEGA_O = 0.57
_OMEGA_T = 0.41
_NO_VALUE = -999.0
_EDGE_W = (1.0, 0.98, 0.97, 0.96, 0.95)

_B, _T, _C, _H, _W = 4, 12, 1, 512, 512
_NUMEL = _B * _C * _H * _W
_K90_LO = 943717
_K13 = 349525

_IMIN = -2147483648


def _f32_key(x):
    u = jax.lax.bitcast_convert_type(x, jnp.int32)
    return jnp.where(u >= 0, u, _IMIN - u)


def _key_to_f32(k):
    u = jnp.where(k >= 0, k, _IMIN - k)
    return jax.lax.bitcast_convert_type(u.astype(jnp.int32), jnp.float32)


def _select_rank(key, k90, k13):
    imin = jnp.int32(_IMIN)
    imax = jnp.int32(2147483647)
    kmin = jnp.min(key)
    kmax = jnp.max(key)
    n = jnp.int32(_NUMEL)

    def init(k):
        return (kmin - 1, kmax, jnp.int32(0), n,
                _key_to_f32(kmin), _key_to_f32(kmax))

    def done(st, k):
        lo, hi, clo, chi, _, _ = st
        return (chi == k + 1) | (clo == k) | ((hi - lo) == 1)

    def midpoint(st):
        lo, hi, _, _, _, _ = st
        gap = hi - lo
        half = jax.lax.shift_right_logical(gap, 1)
        return jax.lax.bitwise_xor(
            jax.lax.bitwise_xor(lo, imin) + half, imin)

    def secant(st, k, r, guess):
        lo, hi, clo, chi, vlo, vhi = st
        frac = (jnp.float32(k + 1) - clo.astype(jnp.float32)) / (
            chi.astype(jnp.float32) - clo.astype(jnp.float32))
        tv = jnp.where(r == 0, guess, vlo + (vhi - vlo) * frac)
        tu = jax.lax.bitcast_convert_type(tv, jnp.int32)
        tk = jnp.where(tu >= 0, tu, imin - tu)
        return jnp.minimum(jnp.maximum(tk, lo + 1), hi - 1)

    def update(st, k, t, cnt):
        lo, hi, clo, chi, vlo, vhi = st
        act = jnp.logical_not(done(st, k))
        up = act & (cnt >= k + 1)
        dn = act & (cnt < k + 1)
        tval = _key_to_f32(t)
        return (jnp.where(dn, t, lo), jnp.where(up, t, hi),
                jnp.where(dn, cnt, clo), jnp.where(up, cnt, chi),
                jnp.where(dn, tval, vlo), jnp.where(up, tval, vhi))

    def round_pair(sts, t90, t13):
        st90, st13 = sts
        c90 = jnp.sum((key <= t90).astype(jnp.int32))
        c13 = jnp.sum((key <= t13).astype(jnp.int32))
        return (update(st90, k90, t90, c90), update(st13, k13, t13, c13))

    def secant_body(r, sts):
        t90 = secant(sts[0], k90, r, jnp.float32(1.2815516))
        t13 = secant(sts[1], k13, r, jnp.float32(-0.4307273))
        return round_pair(sts, t90, t13)

    def mid_body(r, sts):
        return round_pair(sts, midpoint(sts[0]), midpoint(sts[1]))

    sts = (init(k90), init(k13))
    sts = jax.lax.fori_loop(0, 8, secant_body, sts)
    both = lambda s: done(s[0], k90) & done(s[1], k13)
    sts = jax.lax.cond(both(sts), lambda s: s,
                       lambda s: jax.lax.fori_loop(0, 12, mid_body, s), sts)
    sts = jax.lax.cond(both(sts), lambda s: s,
                       lambda s: jax.lax.fori_loop(0, 32, mid_body, s), sts)
    st90, st13 = sts

    def extract(st, k):
        lo, hi, clo, chi, _, _ = st
        m_gt = jnp.min(jnp.where(key > lo, key, imax))
        m_le = jnp.max(jnp.where(key <= hi, key, imin))
        return jnp.where((hi - lo) == 1, hi,
                         jnp.where(clo == k, m_gt, m_le))

    return extract(st90, k90), extract(st13, k13)


def _edge_weight(idx):
    e = jnp.zeros_like(idx, dtype=jnp.float32)
    for i, w in enumerate(_EDGE_W):
        e = e + jnp.where(idx == i, w, 0.0) + jnp.where(idx == (_H - 1 - i), w, 0.0)
    return e


def _min_weight(m):
    e = jnp.zeros_like(m, dtype=jnp.float32)
    for i, w in enumerate(_EDGE_W):
        e = e + jnp.where(m == i, w, 0.0)
    return e


def _loss_kernel(yp_ref, yt_ref, out_ref):
    yt = yt_ref[...]
    yp = yp_ref[...]

    key = _f32_key(yt)
    key90a, key13 = _select_rank(key, _K90_LO, _K13)
    c_a = jnp.sum((key <= key90a).astype(jnp.int32))
    nxt = jnp.min(jnp.where(key > key90a, key, jnp.int32(2147483647)))
    key90b = jnp.where(c_a >= _K90_LO + 2, key90a, nxt)
    va = _key_to_f32(key90a)
    vb = _key_to_f32(key90b)
    q90 = va + (vb - va) * jnp.float32(0.5)
    q13 = _key_to_f32(key13)

    h = jax.lax.broadcasted_iota(jnp.int32, yt.shape, 3)
    w = jax.lax.broadcasted_iota(jnp.int32, yt.shape, 4)
    diag = (h == w) | (h + w == _H - 1)
    bmask = _edge_weight(h) + _edge_weight(w) + jnp.where(
        diag, _min_weight(jnp.minimum(h, _H - 1 - h)), 0.0)

    diff = jnp.abs(yt - yp)
    no_value = yt == _NO_VALUE
    outlier = yt > q90
    normal = jnp.logical_not(no_value | outlier)
    over = (yp >= yt) & normal
    under = (yp < yt) & normal
    torr = (yt >= q13) & normal
    wts = _ALPHA * jnp.exp(_BETA * yt)
    sq = (yt - yp) * (yt - yp)
    wsq = wts * sq

    zero = jnp.float32(0.0)
    s_low = jnp.sum(jnp.where(no_value, diff, zero))
    s_out = jnp.sum(jnp.where(outlier, diff, zero))
    s_bnd = jnp.sum(bmask * diff)
    s_over = jnp.sum(jnp.where(over, diff, zero))
    s_under = jnp.sum(jnp.where(under, diff, zero))
    s_tover = jnp.sum(jnp.where(torr & over, wsq, zero))
    s_tunder = jnp.sum(jnp.where(torr & under, wsq, zero))
    s_abs = jnp.sum(jnp.where(no_value, zero, diff))
    s_cnt = jnp.sum(jnp.where(no_value, zero, jnp.float32(1.0)))

    inv_n = jnp.float32(1.0 / _NUMEL)
    loss_t = (
        _OMEGA_O * s_low
        + (1.0 - _OMEGA_O) * s_out
        + (1.0 - _OMEGA_O) * s_bnd
        + (1.0 - _OMEGA_O) * s_over + _OMEGA_O * s_under
        + (1.0 - _OMEGA_T) * s_tover + _OMEGA_T * s_tunder
    ) * inv_n

    r = jax.lax.broadcasted_iota(jnp.int32, (1, 8, 128), 1)
    c = jax.lax.broadcasted_iota(jnp.int32, (1, 8, 128), 2)
    first = r == 0
    tile = (jnp.where(first & (c == 0), loss_t, zero)
            + jnp.where(first & (c == 1), s_abs, zero)
            + jnp.where(first & (c == 2), s_cnt, zero))
    out_ref[...] = tile


@jax.jit
def kernel(y_pred, y_true):
    block = (_B, 1, _C, _H, _W)
    partials = pl.pallas_call(
        _loss_kernel,
        grid=(_T,),
        in_specs=[
            pl.BlockSpec(block, lambda t: (0, t, 0, 0, 0)),
            pl.BlockSpec(block, lambda t: (0, t, 0, 0, 0)),
        ],
        out_specs=pl.BlockSpec((1, 8, 128), lambda t: (t, 0, 0)),
        out_shape=jax.ShapeDtypeStruct((_T, 8, 128), jnp.float32),
    )(y_pred, y_true)

    losses = partials[:, 0, 0]
    s_abs = partials[:, 0, 1]
    s_cnt = partials[:, 0, 2]
    seasons = ((0, 1, 11), (2, 3, 4), (5, 6, 7), (8, 9, 10))
    seasonal = jnp.float32(0.0)
    for idx in seasons:
        ii = jnp.asarray(idx)
        seasonal = seasonal + jnp.sum(s_abs[ii]) / jnp.sum(s_cnt[ii])
    return jnp.mean(losses) + seasonal


# speedup vs baseline: 2.7648x; 1.0881x over previous
import jax
import jax.numpy as jnp
from jax.experimental import pallas as pl

_ALPHA = 0.007
_BETA = 0.016
_OMEGA_O = 0.57
_OMEGA_T = 0.41
_NO_VALUE = -999.0
_EDGE_W = (1.0, 0.98, 0.97, 0.96, 0.95)

_B, _T, _C, _H, _W = 4, 12, 1, 512, 512
_NUMEL = _B * _C * _H * _W
_K90_LO = 943717
_K13 = 349525

_IMIN = -2147483648
_IMAX = 2147483647


def _key_of(v):
    u = jax.lax.bitcast_convert_type(v, jnp.int32)
    return jnp.where(u >= 0, u, _IMIN - u)


def _val_of(k):
    u = jnp.where(k >= 0, k, _IMIN - k)
    return jax.lax.bitcast_convert_type(u.astype(jnp.int32), jnp.float32)


def _select_rank(yt, k90, k13):
    ninf = jnp.float32(-jnp.inf)
    pinf = jnp.float32(jnp.inf)
    imin = jnp.int32(_IMIN)
    n = jnp.int32(_NUMEL)
    klo0 = _key_of(ninf)
    khi0 = _key_of(pinf)

    def init():
        return (klo0, khi0, jnp.int32(0), n,
                jnp.float32(-6.5), jnp.float32(6.5))

    def done(st, k):
        lo, hi, clo, chi, _, _ = st
        return (chi == k + 1) | (clo == k) | ((hi - lo) == 1)

    def midpoint(st):
        lo, hi = st[0], st[1]
        gap = hi - lo
        half = jax.lax.shift_right_logical(gap, 1)
        return jax.lax.bitwise_xor(
            jax.lax.bitwise_xor(lo, imin) + half, imin)

    def secant(st, k, r, guess):
        lo, hi, clo, chi, vlo, vhi = st
        frac = (jnp.float32(k + 1) - clo.astype(jnp.float32)) / (
            chi.astype(jnp.float32) - clo.astype(jnp.float32))
        tv = jnp.where(r == 0, guess, vlo + (vhi - vlo) * frac)
        return jnp.minimum(jnp.maximum(_key_of(tv), lo + 1), hi - 1)

    def update(st, k, t, cnt):
        lo, hi, clo, chi, vlo, vhi = st
        act = jnp.logical_not(done(st, k))
        up = act & (cnt >= k + 1)
        dn = act & (cnt < k + 1)
        tval = _val_of(t)
        return (jnp.where(dn, t, lo), jnp.where(up, t, hi),
                jnp.where(dn, cnt, clo), jnp.where(up, cnt, chi),
                jnp.where(dn, tval, vlo), jnp.where(up, tval, vhi))

    def round_pair(sts, t90, t13):
        st90, st13 = sts
        c90 = jnp.sum((yt <= _val_of(t90)).astype(jnp.int32))
        c13 = jnp.sum((yt <= _val_of(t13)).astype(jnp.int32))
        return (update(st90, k90, t90, c90), update(st13, k13, t13, c13))

    def secant_body(r, sts):
        t90 = secant(sts[0], k90, r, jnp.float32(1.2815516))
        t13 = secant(sts[1], k13, r, jnp.float32(-0.4307273))
        return round_pair(sts, t90, t13)

    def mid_body(r, sts):
        return round_pair(sts, midpoint(sts[0]), midpoint(sts[1]))

    sts = (init(), init())
    sts = jax.lax.fori_loop(0, 8, secant_body, sts)
    both = lambda s: done(s[0], k90) & done(s[1], k13)
    sts = jax.lax.cond(both(sts), lambda s: s,
                       lambda s: jax.lax.fori_loop(0, 12, mid_body, s), sts)
    sts = jax.lax.cond(both(sts), lambda s: s,
                       lambda s: jax.lax.fori_loop(0, 32, mid_body, s), sts)
    st90, st13 = sts

    def extract(st, k):
        lo, hi, clo, chi, _, _ = st
        m_gt = jnp.min(jnp.where(yt > _val_of(lo), yt, pinf))
        m_le = jnp.max(jnp.where(yt <= _val_of(hi), yt, ninf))
        return jnp.where((hi - lo) == 1, _val_of(hi),
                         jnp.where(clo == k, m_gt, m_le))

    return extract(st90, k90), extract(st13, k13)


def _edge_weight(idx):
    e = jnp.zeros_like(idx, dtype=jnp.float32)
    for i, w in enumerate(_EDGE_W):
        e = e + jnp.where(idx == i, w, 0.0) + jnp.where(idx == (_H - 1 - i), w, 0.0)
    return e


def _min_weight(m):
    e = jnp.zeros_like(m, dtype=jnp.float32)
    for i, w in enumerate(_EDGE_W):
        e = e + jnp.where(m == i, w, 0.0)
    return e


def _loss_kernel(yp_ref, yt_ref, out_ref):
    yt = yt_ref[...]
    yp = yp_ref[...]

    va, q13 = _select_rank(yt, _K90_LO, _K13)
    c_a = jnp.sum((yt <= va).astype(jnp.int32))
    nxt = jnp.min(jnp.where(yt > va, yt, jnp.float32(jnp.inf)))
    vb = jnp.where(c_a >= _K90_LO + 2, va, nxt)
    q90 = va + (vb - va) * jnp.float32(0.5)

    h = jax.lax.broadcasted_iota(jnp.int32, yt.shape, 3)
    w = jax.lax.broadcasted_iota(jnp.int32, yt.shape, 4)
    diag = (h == w) | (h + w == _H - 1)
    bmask = _edge_weight(h) + _edge_weight(w) + jnp.where(
        diag, _min_weight(jnp.minimum(h, _H - 1 - h)), 0.0)

    diff = jnp.abs(yt - yp)
    no_value = yt == _NO_VALUE
    outlier = yt > q90
    normal = jnp.logical_not(no_value | outlier)
    over = yp >= yt
    torr = (yt >= q13) & normal
    wsq = (_ALPHA * jnp.exp(_BETA * yt)) * ((yt - yp) * (yt - yp))

    zero = jnp.float32(0.0)
    om_o = jnp.float32(_OMEGA_O)
    om_o1 = jnp.float32(1.0 - _OMEGA_O)
    om_t = jnp.float32(_OMEGA_T)
    om_t1 = jnp.float32(1.0 - _OMEGA_T)
    coef_d = (jnp.where(no_value, om_o, zero)
              + jnp.where(outlier, om_o1, zero)
              + om_o1 * bmask
              + jnp.where(normal, jnp.where(over, om_o1, om_o), zero))
    coef_w = jnp.where(torr, jnp.where(over, om_t1, om_t), zero)
    loss_sum = jnp.sum(coef_d * diff + coef_w * wsq)
    s_abs = jnp.sum(jnp.where(no_value, zero, diff))
    s_cnt = jnp.sum(jnp.where(no_value, zero, jnp.float32(1.0)))
    loss_t = loss_sum * jnp.float32(1.0 / _NUMEL)

    r = jax.lax.broadcasted_iota(jnp.int32, (1, 8, 128), 1)
    c = jax.lax.broadcasted_iota(jnp.int32, (1, 8, 128), 2)
    first = r == 0
    tile = (jnp.where(first & (c == 0), loss_t, zero)
            + jnp.where(first & (c == 1), s_abs, zero)
            + jnp.where(first & (c == 2), s_cnt, zero))
    out_ref[...] = tile


@jax.jit
def kernel(y_pred, y_true):
    block = (_B, 1, _C, _H, _W)
    partials = pl.pallas_call(
        _loss_kernel,
        grid=(_T,),
        in_specs=[
            pl.BlockSpec(block, lambda t: (0, t, 0, 0, 0)),
            pl.BlockSpec(block, lambda t: (0, t, 0, 0, 0)),
        ],
        out_specs=pl.BlockSpec((1, 8, 128), lambda t: (t, 0, 0)),
        out_shape=jax.ShapeDtypeStruct((_T, 8, 128), jnp.float32),
    )(y_pred, y_true)

    losses = partials[:, 0, 0]
    s_abs = partials[:, 0, 1]
    s_cnt = partials[:, 0, 2]
    seasons = ((0, 1, 11), (2, 3, 4), (5, 6, 7), (8, 9, 10))
    seasonal = jnp.float32(0.0)
    for idx in seasons:
        ii = jnp.asarray(idx)
        seasonal = seasonal + jnp.sum(s_abs[ii]) / jnp.sum(s_cnt[ii])
    return jnp.mean(losses) + seasonal


# speedup vs baseline: 2.8886x; 1.0448x over previous
import jax
import jax.numpy as jnp
from jax.experimental import pallas as pl

_ALPHA = 0.007
_BETA = 0.016
_OMEGA_O = 0.57
_OMEGA_T = 0.41
_NO_VALUE = -999.0
_EDGE_W = (1.0, 0.98, 0.97, 0.96, 0.95)

_B, _T, _C, _H, _W = 4, 12, 1, 512, 512
_NUMEL = _B * _C * _H * _W
_K90_LO = 943717
_K13 = 349525

_IMIN = -2147483648
_IMAX = 2147483647


def _key_of(v):
    u = jax.lax.bitcast_convert_type(v, jnp.int32)
    return jnp.where(u >= 0, u, _IMIN - u)


def _val_of(k):
    u = jnp.where(k >= 0, k, _IMIN - k)
    return jax.lax.bitcast_convert_type(u.astype(jnp.int32), jnp.float32)


def _select_rank(yt, k90, k13):
    ninf = jnp.float32(-jnp.inf)
    pinf = jnp.float32(jnp.inf)
    imin = jnp.int32(_IMIN)
    n = jnp.int32(_NUMEL)
    klo0 = _key_of(ninf)
    khi0 = _key_of(pinf)

    def init():
        return (klo0, khi0, jnp.int32(0), n,
                jnp.float32(-6.5), jnp.float32(6.5))

    def done(st, k):
        lo, hi, clo, chi, _, _ = st
        return (chi == k + 1) | (clo == k) | ((hi - lo) == 1)

    def midpoint(st):
        lo, hi = st[0], st[1]
        gap = hi - lo
        half = jax.lax.shift_right_logical(gap, 1)
        return jax.lax.bitwise_xor(
            jax.lax.bitwise_xor(lo, imin) + half, imin)

    def secant(st, k, r, guess):
        lo, hi, clo, chi, vlo, vhi = st
        frac = (jnp.float32(k + 1) - clo.astype(jnp.float32)) / (
            chi.astype(jnp.float32) - clo.astype(jnp.float32))
        tv = jnp.where(r == 0, guess, vlo + (vhi - vlo) * frac)
        return jnp.minimum(jnp.maximum(_key_of(tv), lo + 1), hi - 1)

    def update(st, k, t, cnt):
        lo, hi, clo, chi, vlo, vhi = st
        act = jnp.logical_not(done(st, k))
        up = act & (cnt >= k + 1)
        dn = act & (cnt < k + 1)
        tval = _val_of(t)
        return (jnp.where(dn, t, lo), jnp.where(up, t, hi),
                jnp.where(dn, cnt, clo), jnp.where(up, cnt, chi),
                jnp.where(dn, tval, vlo), jnp.where(up, tval, vhi))

    def round_pair(sts, t90, t13):
        st90, st13 = sts
        c90 = jnp.sum((yt <= _val_of(t90)).astype(jnp.int32))
        c13 = jnp.sum((yt <= _val_of(t13)).astype(jnp.int32))
        return (update(st90, k90, t90, c90), update(st13, k13, t13, c13))

    def secant_body(r, sts):
        t90 = secant(sts[0], k90, r, jnp.float32(1.2815516))
        t13 = secant(sts[1], k13, r, jnp.float32(-0.4307273))
        return round_pair(sts, t90, t13)

    def mid_body(r, sts):
        return round_pair(sts, midpoint(sts[0]), midpoint(sts[1]))

    sts = (init(), init())
    sts = jax.lax.fori_loop(0, 7, secant_body, sts)
    both = lambda s: done(s[0], k90) & done(s[1], k13)
    sts = jax.lax.cond(both(sts), lambda s: s,
                       lambda s: jax.lax.fori_loop(0, 8, mid_body, s), sts)
    sts = jax.lax.cond(both(sts), lambda s: s,
                       lambda s: jax.lax.fori_loop(0, 32, mid_body, s), sts)
    st90, st13 = sts

    def extract(st, k):
        lo, hi, clo, chi, _, _ = st
        m_gt = jnp.min(jnp.where(yt > _val_of(lo), yt, pinf))
        m_le = jnp.max(jnp.where(yt <= _val_of(hi), yt, ninf))
        return jnp.where((hi - lo) == 1, _val_of(hi),
                         jnp.where(clo == k, m_gt, m_le))

    return extract(st90, k90), extract(st13, k13)


def _edge_weight(idx):
    e = jnp.zeros_like(idx, dtype=jnp.float32)
    for i, w in enumerate(_EDGE_W):
        e = e + jnp.where(idx == i, w, 0.0) + jnp.where(idx == (_H - 1 - i), w, 0.0)
    return e


def _min_weight(m):
    e = jnp.zeros_like(m, dtype=jnp.float32)
    for i, w in enumerate(_EDGE_W):
        e = e + jnp.where(m == i, w, 0.0)
    return e


def _loss_kernel(yp_ref, yt_ref, out_ref):
    yt = yt_ref[...]
    yp = yp_ref[...]

    va, q13 = _select_rank(yt, _K90_LO, _K13)
    c_a = jnp.sum((yt <= va).astype(jnp.int32))
    nxt = jnp.min(jnp.where(yt > va, yt, jnp.float32(jnp.inf)))
    vb = jnp.where(c_a >= _K90_LO + 2, va, nxt)
    q90 = va + (vb - va) * jnp.float32(0.5)

    h = jax.lax.broadcasted_iota(jnp.int32, yt.shape, 3)
    w = jax.lax.broadcasted_iota(jnp.int32, yt.shape, 4)
    diag = (h == w) | (h + w == _H - 1)
    bmask = _edge_weight(h) + _edge_weight(w) + jnp.where(
        diag, _min_weight(jnp.minimum(h, _H - 1 - h)), 0.0)

    diff = jnp.abs(yt - yp)
    no_value = yt == _NO_VALUE
    outlier = yt > q90
    normal = jnp.logical_not(no_value | outlier)
    over = yp >= yt
    torr = (yt >= q13) & normal
    wsq = (_ALPHA * jnp.exp(_BETA * yt)) * ((yt - yp) * (yt - yp))

    zero = jnp.float32(0.0)
    om_o = jnp.float32(_OMEGA_O)
    om_o1 = jnp.float32(1.0 - _OMEGA_O)
    om_t = jnp.float32(_OMEGA_T)
    om_t1 = jnp.float32(1.0 - _OMEGA_T)
    coef_d = (jnp.where(no_value, om_o, zero)
              + jnp.where(outlier, om_o1, zero)
              + om_o1 * bmask
              + jnp.where(normal, jnp.where(over, om_o1, om_o), zero))
    coef_w = jnp.where(torr, jnp.where(over, om_t1, om_t), zero)
    loss_sum = jnp.sum(coef_d * diff + coef_w * wsq)
    s_abs = jnp.sum(jnp.where(no_value, zero, diff))
    s_cnt = jnp.sum(jnp.where(no_value, zero, jnp.float32(1.0)))
    loss_t = loss_sum * jnp.float32(1.0 / _NUMEL)

    r = jax.lax.broadcasted_iota(jnp.int32, (1, 8, 128), 1)
    c = jax.lax.broadcasted_iota(jnp.int32, (1, 8, 128), 2)
    first = r == 0
    tile = (jnp.where(first & (c == 0), loss_t, zero)
            + jnp.where(first & (c == 1), s_abs, zero)
            + jnp.where(first & (c == 2), s_cnt, zero))
    out_ref[...] = tile


@jax.jit
def kernel(y_pred, y_true):
    block = (_B, 1, _C, _H, _W)
    partials = pl.pallas_call(
        _loss_kernel,
        grid=(_T,),
        in_specs=[
            pl.BlockSpec(block, lambda t: (0, t, 0, 0, 0)),
            pl.BlockSpec(block, lambda t: (0, t, 0, 0, 0)),
        ],
        out_specs=pl.BlockSpec((1, 8, 128), lambda t: (t, 0, 0)),
        out_shape=jax.ShapeDtypeStruct((_T, 8, 128), jnp.float32),
    )(y_pred, y_true)

    losses = partials[:, 0, 0]
    s_abs = partials[:, 0, 1]
    s_cnt = partials[:, 0, 2]
    seasons = ((0, 1, 11), (2, 3, 4), (5, 6, 7), (8, 9, 10))
    seasonal = jnp.float32(0.0)
    for idx in seasons:
        ii = jnp.asarray(idx)
        seasonal = seasonal + jnp.sum(s_abs[ii]) / jnp.sum(s_cnt[ii])
    return jnp.mean(losses) + seasonal
